# baseline (speedup 1.0000x reference)
import functools

import jax
import jax.numpy as jnp
from jax import lax
from jax.experimental import pallas as pl
from jax.experimental.pallas import tpu as pltpu
from jax.experimental.pallas import tpu_sc as plsc

DIM = 64
B = 16384
L = 50
LP = 64
BL = B * L
BLP = B * LP
NC, NS = 2, 16
NW = NC * NS
EPW = B // NW
CB = 16
NCH = EPW // CB
SLOTS = CB * LP
SUB = 128
NSUB = SLOTS // SUB

_sc_mesh = plsc.VectorSubcoreMesh(core_axis_name="c", subcore_axis_name="s")


@functools.partial(
    pl.kernel,
    out_type=jax.ShapeDtypeStruct((2 * BL, DIM), jnp.float32),
    mesh=_sc_mesh,
    scratch_types=[
        pltpu.VMEM((SLOTS,), jnp.int32),
        pltpu.VMEM((SLOTS, DIM), jnp.float32),
        pltpu.SemaphoreType.DMA,
        pltpu.SemaphoreType.DMA,
    ],
    compiler_params=pltpu.CompilerParams(use_tc_tiling_on_sc=False),
)
def _sc_gather(semb, temb, xidx, yidx, out, idx_v, rows_v, gsem, osem):
    wid = lax.axis_index("s") * NC + lax.axis_index("c")

    def do_side(table, idx_hbm, out_base):
        def chunk_body(c, carry):
            ex0 = wid * EPW + c * CB
            pltpu.sync_copy(idx_hbm.at[pl.ds(ex0 * LP, SLOTS)], idx_v)
            gds = []
            for j in range(NSUB):
                gds.append(
                    pltpu.async_copy(
                        table.at[idx_v.at[pl.ds(j * SUB, SUB)]],
                        rows_v.at[pl.ds(j * SUB, SUB)],
                        gsem,
                    )
                )
            ods = []
            for j in range(NSUB):
                gds[j].wait()
                for b in (2 * j, 2 * j + 1):
                    ods.append(
                        pltpu.async_copy(
                            rows_v.at[pl.ds(b * LP, L)],
                            out.at[pl.ds(out_base + (ex0 + b) * L, L)],
                            osem,
                        )
                    )
            for d in ods:
                d.wait()
            return carry

        lax.fori_loop(0, NCH, chunk_body, 0)

    do_side(semb, xidx, 0)
    do_side(temb, yidx, BL)


_MM_ROWS = 12800


def _mm_body(x_ref, w_ref, o_ref):
    o_ref[...] = lax.dot_general(
        x_ref[...], w_ref[...], (((1,), (0,)), ((), ())),
        preferred_element_type=jnp.float32,
    )


def _project(gathered2, w2):
    return pl.pallas_call(
        _mm_body,
        grid=(BL // _MM_ROWS,),
        in_specs=[
            pl.BlockSpec((_MM_ROWS, 2 * DIM), lambda i: (i, 0)),
            pl.BlockSpec((2 * DIM, 2 * DIM), lambda i: (0, 0)),
        ],
        out_specs=pl.BlockSpec((_MM_ROWS, 2 * DIM), lambda i: (i, 0)),
        out_shape=jax.ShapeDtypeStruct((BL, 2 * DIM), jnp.float32),
    )(gathered2, w2)


def kernel(W_m, semb_table, temb_table, x_idx, y_idx):
    xr = jnp.pad(x_idx.astype(jnp.int32), ((0, 0), (0, LP - L))).reshape(BLP)
    yr = jnp.pad(y_idx.astype(jnp.int32), ((0, 0), (0, LP - L))).reshape(BLP)
    gathered = _sc_gather(semb_table, temb_table, xr, yr)
    wt = W_m.T
    z = jnp.zeros((DIM, DIM), jnp.float32)
    w2 = jnp.block([[wt, z], [z, wt]])
    proj = _project(gathered.reshape(BL, 2 * DIM), w2)
    return proj.reshape(2, B, L, DIM)


# speedup vs baseline: 4.4718x; 4.4718x over previous
import functools

import jax
import jax.numpy as jnp
from jax import lax
from jax.experimental import pallas as pl
from jax.experimental.pallas import tpu as pltpu
from jax.experimental.pallas import tpu_sc as plsc

DIM = 64
B = 16384
L = 50
BL = B * L
NC, NS = 2, 16
NW = NC * NS
RPW = BL // NW
SUB = 128
CHUNK = 1280
NSUB = CHUNK // SUB
NCH = RPW // CHUNK

_sc_mesh = plsc.VectorSubcoreMesh(core_axis_name="c", subcore_axis_name="s")


@functools.partial(
    pl.kernel,
    out_type=jax.ShapeDtypeStruct((2 * BL, DIM), jnp.float32),
    mesh=_sc_mesh,
    scratch_types=[
        pltpu.VMEM((CHUNK,), jnp.int32),
        pltpu.VMEM((CHUNK, DIM), jnp.float32),
        pltpu.SemaphoreType.DMA,
    ],
    compiler_params=pltpu.CompilerParams(use_tc_tiling_on_sc=False),
)
def _sc_gather(semb, temb, xidx, yidx, out, idx_v, rows_v, sem):
    wid = lax.axis_index("s") * NC + lax.axis_index("c")
    base = wid * RPW

    def do_side(table, idx_hbm, out_base):
        def chunk_body(c, carry):
            off = base + c * CHUNK
            pltpu.sync_copy(idx_hbm.at[pl.ds(off, CHUNK)], idx_v)
            descs = []
            for j in range(NSUB):
                descs.append(
                    pltpu.async_copy(
                        table.at[idx_v.at[pl.ds(j * SUB, SUB)]],
                        rows_v.at[pl.ds(j * SUB, SUB)],
                        sem,
                    )
                )
            for d in descs:
                d.wait()
            pltpu.sync_copy(rows_v, out.at[pl.ds(out_base + off, CHUNK)])
            return carry

        lax.fori_loop(0, NCH, chunk_body, 0)

    do_side(semb, xidx, 0)
    do_side(temb, yidx, BL)


_MM_ROWS = 12800


def _mm_body(x_ref, w_ref, o_ref):
    o_ref[...] = lax.dot_general(
        x_ref[...], w_ref[...], (((1,), (0,)), ((), ())),
        preferred_element_type=jnp.float32,
    )


def _project(gathered2, w2):
    return pl.pallas_call(
        _mm_body,
        grid=(BL // _MM_ROWS,),
        in_specs=[
            pl.BlockSpec((_MM_ROWS, 2 * DIM), lambda i: (i, 0)),
            pl.BlockSpec((2 * DIM, 2 * DIM), lambda i: (0, 0)),
        ],
        out_specs=pl.BlockSpec((_MM_ROWS, 2 * DIM), lambda i: (i, 0)),
        out_shape=jax.ShapeDtypeStruct((BL, 2 * DIM), jnp.float32),
    )(gathered2, w2)


def kernel(W_m, semb_table, temb_table, x_idx, y_idx):
    xr = x_idx.astype(jnp.int32).reshape(BL)
    yr = y_idx.astype(jnp.int32).reshape(BL)
    gathered = _sc_gather(semb_table, temb_table, xr, yr)
    wt = W_m.T
    z = jnp.zeros((DIM, DIM), jnp.float32)
    w2 = jnp.block([[wt, z], [z, wt]])
    proj = _project(gathered.reshape(BL, 2 * DIM), w2)
    return proj.reshape(2, B, L, DIM)
